# Initial kernel scaffold; baseline (speedup 1.0000x reference)
import jax
import jax.numpy as jnp
from jax.experimental import pallas as pl


def kernel(queries, keys, values, Wq, bq, Wk, bk, Wv, bv, Wo, bo):
    raise NotImplementedError("write your pallas kernel here")


# speedup vs baseline: 3.8185x; 3.8185x over previous
import functools

import jax
import jax.numpy as jnp
import numpy as np
from jax import lax
from jax.experimental import pallas as pl
from jax.experimental.pallas import tpu as pltpu

B = 2
L = 2048
D_MODEL = 1024
N_HEADS = 16
D_K = D_MODEL // N_HEADS
TOP_K = 8
BH = B * N_HEADS

_t = np.arange(L, dtype=np.float64)
_theta = (2.0 * np.pi / L) * np.outer(_t, _t)
_FC = np.cos(_theta).astype(np.float32)
_FS = np.sin(_theta).astype(np.float32)
_ED = np.zeros((B * D_MODEL, BH), dtype=np.float32)
for _c in range(B * D_MODEL):
    _ED[_c, _c // D_K] = 1.0 / (L * D_K)


def _mm_kernel(x_ref, y_ref, o_ref, *, precision):
    @pl.when(pl.program_id(2) == 0)
    def _():
        o_ref[...] = jnp.zeros_like(o_ref)

    o_ref[...] += jnp.dot(x_ref[...], y_ref[...],
                          preferred_element_type=jnp.float32,
                          precision=precision)


def _mm_bias_kernel(x_ref, y_ref, b_ref, o_ref, *, precision):
    @pl.when(pl.program_id(2) == 0)
    def _():
        o_ref[...] = jnp.broadcast_to(b_ref[...], o_ref.shape)

    o_ref[...] += jnp.dot(x_ref[...], y_ref[...],
                          preferred_element_type=jnp.float32,
                          precision=precision)


def _matmul(x, y, bias=None, bm=512, bn=512, bk=512,
            precision=lax.Precision.HIGHEST):
    M, K = x.shape
    _, N = y.shape
    bm, bn, bk = min(bm, M), min(bn, N), min(bk, K)
    grid = (M // bm, N // bn, K // bk)
    in_specs = [
        pl.BlockSpec((bm, bk), lambda i, j, k: (i, k)),
        pl.BlockSpec((bk, bn), lambda i, j, k: (k, j)),
    ]
    args = [x, y]
    if bias is None:
        body = functools.partial(_mm_kernel, precision=precision)
    else:
        body = functools.partial(_mm_bias_kernel, precision=precision)
        in_specs.append(pl.BlockSpec((1, bn), lambda i, j, k: (0, j)))
        args.append(bias.reshape(1, N))
    return pl.pallas_call(
        body,
        grid=grid,
        in_specs=in_specs,
        out_specs=pl.BlockSpec((bm, bn), lambda i, j, k: (i, j)),
        out_shape=jax.ShapeDtypeStruct((M, N), jnp.float32),
        compiler_params=pltpu.CompilerParams(
            dimension_semantics=("parallel", "parallel", "arbitrary")),
    )(*args)


def _spectrum_kernel(qc_ref, qs_ref, kc_ref, ks_ref, ed_ref, sr_ref, si_ref):
    qc, qs = qc_ref[...], qs_ref[...]
    kc, ks = kc_ref[...], ks_ref[...]
    ed = ed_ref[...]
    sr_ref[...] = jnp.dot(qc * kc + qs * ks, ed,
                          preferred_element_type=jnp.float32, precision=lax.Precision.HIGHEST)
    si_ref[...] = jnp.dot(qc * ks - qs * kc, ed,
                          preferred_element_type=jnp.float32, precision=lax.Precision.HIGHEST)


def _cross_spectrum(qc, qs, kc, ks, ed, bm=256):
    grid = (L // bm,)
    bw = B * D_MODEL
    spec = pl.BlockSpec((bm, bw), lambda i: (i, 0))
    return pl.pallas_call(
        _spectrum_kernel,
        grid=grid,
        in_specs=[spec, spec, spec, spec,
                  pl.BlockSpec((bw, BH), lambda i: (0, 0))],
        out_specs=[pl.BlockSpec((bm, BH), lambda i: (i, 0))] * 2,
        out_shape=[jax.ShapeDtypeStruct((L, BH), jnp.float32)] * 2,
        compiler_params=pltpu.CompilerParams(
            dimension_semantics=("parallel",)),
    )(qc, qs, kc, ks, ed)


def _idft_kernel(fc_ref, fs_ref, sr_ref, si_ref, o_ref):
    @pl.when(pl.program_id(1) == 0)
    def _():
        o_ref[...] = jnp.zeros_like(o_ref)

    o_ref[...] += (jnp.dot(fc_ref[...], sr_ref[...],
                           preferred_element_type=jnp.float32, precision=lax.Precision.HIGHEST)
                   - jnp.dot(fs_ref[...], si_ref[...],
                             preferred_element_type=jnp.float32, precision=lax.Precision.HIGHEST))


def _idft(fc, fs, sr, si, bm=512, bk=512):
    grid = (L // bm, L // bk)
    fspec = pl.BlockSpec((bm, bk), lambda i, k: (i, k))
    sspec = pl.BlockSpec((bk, BH), lambda i, k: (k, 0))
    return pl.pallas_call(
        _idft_kernel,
        grid=grid,
        in_specs=[fspec, fspec, sspec, sspec],
        out_specs=pl.BlockSpec((bm, BH), lambda i, k: (i, 0)),
        out_shape=jax.ShapeDtypeStruct((L, BH), jnp.float32),
        compiler_params=pltpu.CompilerParams(
            dimension_semantics=("parallel", "arbitrary")),
    )(fc, fs, sr, si)


def _agg_kernel(corr_ref, v2_ref, o_ref):
    cv = corr_ref[...].reshape(1, L)
    iota = lax.broadcasted_iota(jnp.int32, (1, L), 1)
    vals, idxs = [], []
    for _ in range(TOP_K):
        m = jnp.max(cv)
        idx = jnp.min(jnp.where(cv == m, iota, L))
        vals.append(m)
        idxs.append(idx)
        cv = jnp.where(iota == idx, -jnp.inf, cv)
    exps = [jnp.exp(val - vals[0]) for val in vals]
    total = exps[0]
    for e in exps[1:]:
        total = total + e
    acc = (exps[0] / total) * v2_ref[0, pl.ds(L - idxs[0], L), :]
    for j in range(1, TOP_K):
        acc += (exps[j] / total) * v2_ref[0, pl.ds(L - idxs[j], L), :]
    o_ref[...] = acc.reshape(1, L, D_K)


def _topk_agg(corr3d, v2):
    return pl.pallas_call(
        _agg_kernel,
        grid=(BH,),
        in_specs=[
            pl.BlockSpec((1, 1, L), lambda g: (g, 0, 0)),
            pl.BlockSpec((1, 2 * L, D_K), lambda g: (g, 0, 0)),
        ],
        out_specs=pl.BlockSpec((1, L, D_K), lambda g: (g, 0, 0)),
        out_shape=jax.ShapeDtypeStruct((BH, L, D_K), jnp.float32),
        compiler_params=pltpu.CompilerParams(
            dimension_semantics=("parallel",)),
    )(corr3d, v2)


def kernel(queries, keys, values, Wq, bq, Wk, bk, Wv, bv, Wo, bo):
    fc = jnp.asarray(_FC)
    fs = jnp.asarray(_FS)
    ed = jnp.asarray(_ED)

    q = _matmul(queries.reshape(B * L, D_MODEL), Wq.T, bq, bk=1024,
                precision=lax.Precision.DEFAULT)
    k = _matmul(keys.reshape(B * L, D_MODEL), Wk.T, bk, bk=1024,
                precision=lax.Precision.DEFAULT)
    v = _matmul(values.reshape(B * L, D_MODEL), Wv.T, bv, bk=1024,
                precision=lax.Precision.DEFAULT)

    qf = q.reshape(B, L, D_MODEL).transpose(1, 0, 2).reshape(L, B * D_MODEL)
    kf = k.reshape(B, L, D_MODEL).transpose(1, 0, 2).reshape(L, B * D_MODEL)
    x2 = jnp.concatenate([qf, kf], axis=1)

    tc = _matmul(fc, x2)
    ts = _matmul(fs, x2)
    qc, kc = tc[:, : B * D_MODEL], tc[:, B * D_MODEL:]
    qs, ks = ts[:, : B * D_MODEL], ts[:, B * D_MODEL:]

    sr, si = _cross_spectrum(qc, qs, kc, ks, ed)
    corr = _idft(fc, fs, sr, si)
    corr3d = corr.T.reshape(BH, 1, L)

    vh = v.reshape(B, L, N_HEADS, D_K).transpose(0, 2, 1, 3)
    vh = vh.reshape(BH, L, D_K)
    v2 = jnp.concatenate([vh, vh], axis=1)

    out_heads = _topk_agg(corr3d, v2)
    outf = out_heads.reshape(B, N_HEADS, L, D_K).transpose(0, 2, 1, 3)
    outf = outf.reshape(B * L, D_MODEL)
    out = _matmul(outf, Wo.T, bo, bk=1024, precision=lax.Precision.DEFAULT)
    return out.reshape(B, L, D_MODEL)


# speedup vs baseline: 7.3120x; 1.9149x over previous
import functools

import jax
import jax.numpy as jnp
import numpy as np
from jax import lax
from jax.experimental import pallas as pl
from jax.experimental.pallas import tpu as pltpu

B = 2
L = 2048
D_MODEL = 1024
N_HEADS = 16
D_K = D_MODEL // N_HEADS
TOP_K = 8
BH = B * N_HEADS
NF = L // 2 + 1
FPAD = 1152

_f = np.arange(FPAD, dtype=np.float64)
_t = np.arange(L, dtype=np.float64)
_theta = (2.0 * np.pi / L) * np.outer(_f, _t)
_mask = (_f < NF)[:, None]
_FWC = np.where(_mask, np.cos(_theta), 0.0).astype(np.float32)
_FWS = np.where(_mask, np.sin(_theta), 0.0).astype(np.float32)
_w = np.where((_f >= 1) & (_f <= NF - 2), 2.0, 1.0) * (_f < NF)
_IDC = (_w[:, None] * np.cos(_theta)).astype(np.float32)
_IDS = (_w[:, None] * np.sin(_theta)).astype(np.float32)
_ED = np.zeros((D_MODEL, N_HEADS), dtype=np.float32)
for _c in range(D_MODEL):
    _ED[_c, _c // D_K] = 1.0 / (L * D_K)

_HI = lax.Precision.HIGHEST


def _mm_kernel(x_ref, y_ref, o_ref, *, precision):
    @pl.when(pl.program_id(2) == 0)
    def _():
        o_ref[...] = jnp.zeros_like(o_ref)

    o_ref[...] += jnp.dot(x_ref[...], y_ref[...],
                          preferred_element_type=jnp.float32,
                          precision=precision)


def _mm_bias_kernel(x_ref, y_ref, b_ref, o_ref, *, precision):
    @pl.when(pl.program_id(2) == 0)
    def _():
        o_ref[...] = jnp.broadcast_to(b_ref[...], o_ref.shape)

    o_ref[...] += jnp.dot(x_ref[...], y_ref[...],
                          preferred_element_type=jnp.float32,
                          precision=precision)


def _matmul(x, y, bias=None, bm=512, bn=512, bk=512, precision=_HI):
    M, K = x.shape
    _, N = y.shape
    bm, bn, bk = min(bm, M), min(bn, N), min(bk, K)
    grid = (M // bm, N // bn, K // bk)
    in_specs = [
        pl.BlockSpec((bm, bk), lambda i, j, k: (i, k)),
        pl.BlockSpec((bk, bn), lambda i, j, k: (k, j)),
    ]
    args = [x, y]
    if bias is None:
        body = functools.partial(_mm_kernel, precision=precision)
    else:
        body = functools.partial(_mm_bias_kernel, precision=precision)
        in_specs.append(pl.BlockSpec((1, bn), lambda i, j, k: (0, j)))
        args.append(bias.reshape(1, N))
    return pl.pallas_call(
        body,
        grid=grid,
        in_specs=in_specs,
        out_specs=pl.BlockSpec((bm, bn), lambda i, j, k: (i, j)),
        out_shape=jax.ShapeDtypeStruct((M, N), jnp.float32),
        compiler_params=pltpu.CompilerParams(
            dimension_semantics=("parallel", "parallel", "arbitrary")),
    )(*args)


def _fwd_kernel(fc_ref, fs_ref, q_ref, k_ref,
                qc_ref, qs_ref, kc_ref, ks_ref):
    @pl.when(pl.program_id(3) == 0)
    def _():
        qc_ref[...] = jnp.zeros_like(qc_ref)
        qs_ref[...] = jnp.zeros_like(qs_ref)
        kc_ref[...] = jnp.zeros_like(kc_ref)
        ks_ref[...] = jnp.zeros_like(ks_ref)

    fcb, fsb = fc_ref[...], fs_ref[...]
    qb, kb = q_ref[0], k_ref[0]
    dot = functools.partial(jnp.dot, preferred_element_type=jnp.float32,
                            precision=_HI)
    qc_ref[0] += dot(fcb, qb)
    qs_ref[0] += dot(fsb, qb)
    kc_ref[0] += dot(fcb, kb)
    ks_ref[0] += dot(fsb, kb)


def _fwd_transforms(fwc, fws, q3, k3, bm=384, bn=512, bk=512):
    grid = (B, FPAD // bm, D_MODEL // bn, L // bk)
    fspec = pl.BlockSpec((bm, bk), lambda b, i, j, k: (i, k))
    xspec = pl.BlockSpec((1, bk, bn), lambda b, i, j, k: (b, k, j))
    ospec = pl.BlockSpec((1, bm, bn), lambda b, i, j, k: (b, i, j))
    oshape = jax.ShapeDtypeStruct((B, FPAD, D_MODEL), jnp.float32)
    return pl.pallas_call(
        _fwd_kernel,
        grid=grid,
        in_specs=[fspec, fspec, xspec, xspec],
        out_specs=[ospec] * 4,
        out_shape=[oshape] * 4,
        compiler_params=pltpu.CompilerParams(
            dimension_semantics=("parallel", "parallel", "parallel",
                                 "arbitrary")),
    )(fwc, fws, q3, k3)


def _spectrum_kernel(qc_ref, qs_ref, kc_ref, ks_ref, ed_ref, sr_ref, si_ref):
    qc, qs = qc_ref[0], qs_ref[0]
    kc, ks = kc_ref[0], ks_ref[0]
    ed = ed_ref[...]
    dot = functools.partial(jnp.dot, preferred_element_type=jnp.float32,
                            precision=_HI)
    sr_ref[0] = dot(qc * kc + qs * ks, ed)
    si_ref[0] = dot(qc * ks - qs * kc, ed)


def _cross_spectrum(qc, qs, kc, ks, ed, bm=384):
    grid = (B, FPAD // bm)
    spec = pl.BlockSpec((1, bm, D_MODEL), lambda b, i: (b, i, 0))
    return pl.pallas_call(
        _spectrum_kernel,
        grid=grid,
        in_specs=[spec, spec, spec, spec,
                  pl.BlockSpec((D_MODEL, N_HEADS), lambda b, i: (0, 0))],
        out_specs=[pl.BlockSpec((1, bm, N_HEADS), lambda b, i: (b, i, 0))] * 2,
        out_shape=[jax.ShapeDtypeStruct((B, FPAD, N_HEADS), jnp.float32)] * 2,
        compiler_params=pltpu.CompilerParams(
            dimension_semantics=("parallel", "parallel")),
    )(qc, qs, kc, ks, ed)


def _idft_kernel(srt_ref, sit_ref, idc_ref, ids_ref, o_ref):
    @pl.when(pl.program_id(2) == 0)
    def _():
        o_ref[...] = jnp.zeros_like(o_ref)

    dot = functools.partial(jnp.dot, preferred_element_type=jnp.float32,
                            precision=_HI)
    o_ref[0] += (dot(srt_ref[0], idc_ref[...])
                 - dot(sit_ref[0], ids_ref[...]))


def _idft(srt, sit, idc, ids, bn=512, bk=384):
    grid = (B, L // bn, FPAD // bk)
    sspec = pl.BlockSpec((1, N_HEADS, bk), lambda b, j, k: (b, 0, k))
    fspec = pl.BlockSpec((bk, bn), lambda b, j, k: (k, j))
    return pl.pallas_call(
        _idft_kernel,
        grid=grid,
        in_specs=[sspec, sspec, fspec, fspec],
        out_specs=pl.BlockSpec((1, N_HEADS, bn), lambda b, j, k: (b, 0, j)),
        out_shape=jax.ShapeDtypeStruct((B, N_HEADS, L), jnp.float32),
        compiler_params=pltpu.CompilerParams(
            dimension_semantics=("parallel", "parallel", "arbitrary")),
    )(srt, sit, idc, ids)


def _agg_kernel(corr_ref, v_ref, o_ref, scratch):
    vb = v_ref[0]
    scratch[0:L, :] = vb
    scratch[L:2 * L, :] = vb
    cpair = corr_ref[...].reshape(2, L)
    iota = lax.broadcasted_iota(jnp.int32, (1, L), 1)
    for i in range(2):
        cv = cpair[i:i + 1, :]
        vals, idxs = [], []
        for _ in range(TOP_K):
            m = jnp.max(cv)
            idx = jnp.min(jnp.where(cv == m, iota, L))
            vals.append(m)
            idxs.append(idx)
            cv = jnp.where(iota == idx, -jnp.inf, cv)
        exps = [jnp.exp(val - vals[0]) for val in vals]
        total = exps[0]
        for e in exps[1:]:
            total = total + e
        sl = slice(i * D_K, (i + 1) * D_K)
        acc = (exps[0] / total) * scratch[pl.ds(L - idxs[0], L), sl]
        for j in range(1, TOP_K):
            acc += (exps[j] / total) * scratch[pl.ds(L - idxs[j], L), sl]
        o_ref[0, :, sl] = acc


def _topk_agg(corr4, v3):
    return pl.pallas_call(
        _agg_kernel,
        grid=(B, N_HEADS // 2),
        in_specs=[
            pl.BlockSpec((1, 1, 2, L), lambda b, hp: (b, hp, 0, 0)),
            pl.BlockSpec((1, L, 2 * D_K), lambda b, hp: (b, 0, hp)),
        ],
        out_specs=pl.BlockSpec((1, L, 2 * D_K), lambda b, hp: (b, 0, hp)),
        out_shape=jax.ShapeDtypeStruct((B, L, D_MODEL), jnp.float32),
        scratch_shapes=[pltpu.VMEM((2 * L, 2 * D_K), jnp.float32)],
        compiler_params=pltpu.CompilerParams(
            dimension_semantics=("parallel", "parallel")),
    )(corr4, v3)


def kernel(queries, keys, values, Wq, bq, Wk, bk, Wv, bv, Wo, bo):
    fwc = jnp.asarray(_FWC)
    fws = jnp.asarray(_FWS)
    idc = jnp.asarray(_IDC)
    ids = jnp.asarray(_IDS)
    ed = jnp.asarray(_ED)

    q = _matmul(queries.reshape(B * L, D_MODEL), Wq.T, bq, bk=1024,
                precision=lax.Precision.DEFAULT)
    k = _matmul(keys.reshape(B * L, D_MODEL), Wk.T, bk, bk=1024,
                precision=lax.Precision.DEFAULT)
    v = _matmul(values.reshape(B * L, D_MODEL), Wv.T, bv, bk=1024,
                precision=lax.Precision.DEFAULT)

    q3 = q.reshape(B, L, D_MODEL)
    k3 = k.reshape(B, L, D_MODEL)
    v3 = v.reshape(B, L, D_MODEL)

    qc, qs, kc, ks = _fwd_transforms(fwc, fws, q3, k3)
    sr, si = _cross_spectrum(qc, qs, kc, ks, ed)
    srt = sr.transpose(0, 2, 1)
    sit = si.transpose(0, 2, 1)
    corr = _idft(srt, sit, idc, ids)
    corr4 = corr.reshape(B, N_HEADS // 2, 2, L)

    out = _topk_agg(corr4, v3)
    out = _matmul(out.reshape(B * L, D_MODEL), Wo.T, bo, bk=1024,
                  precision=lax.Precision.DEFAULT)
    return out.reshape(B, L, D_MODEL)
